# baseline (speedup 1.0000x reference)
import functools

import jax
import jax.numpy as jnp
from jax import lax
from jax.experimental import pallas as pl
from jax.experimental.pallas import tpu as pltpu
from jax.experimental.pallas import tpu_sc as plsc

N = 10000
E = 320000
D = 128
DH = D // 2

NC = 2
NS = 16
EPS = E // NS
CHUNK = 160
NCHUNK = EPS // CHUNK

RPS = 632
RPS_LAST = N - 15 * RPS


def _segsum_sc(x, ei):
    mesh = plsc.VectorSubcoreMesh(core_axis_name="c", subcore_axis_name="s")

    @functools.partial(
        pl.kernel,
        mesh=mesh,
        compiler_params=pltpu.CompilerParams(use_tc_tiling_on_sc=False),
        out_type=jax.ShapeDtypeStruct((2 * N, DH), jnp.float32),
        scratch_types=[
            pltpu.VMEM((4, 2, CHUNK), jnp.int32),
            pltpu.VMEM((2, CHUNK, DH), jnp.float32),
            pltpu.VMEM_SHARED((N, DH), jnp.float32),
            pltpu.VMEM_SHARED((N, DH), jnp.float32),
            pltpu.SemaphoreType.DMA,
            pltpu.SemaphoreType.DMA,
        ],
    )
    def k(x_hbm, ei_hbm, out_hbm, ibuf, rows, xspm, acc, sem_i, sem_g):
        cid = lax.axis_index("c")
        sid = lax.axis_index("s")
        ebase = sid * EPS

        zeros16 = jnp.zeros((16,), jnp.float32)

        def zero_body(t, _):
            rows[0, t // (DH // 16), pl.ds((t % (DH // 16)) * 16, 16)] = zeros16
            return _

        lax.fori_loop(0, CHUNK * (DH // 16), zero_body, None)
        row0 = pl.multiple_of(sid * RPS, 8)

        def zero_acc(base, total):
            for off in range(0, total, CHUNK):
                size = min(CHUNK, total - off)
                pltpu.sync_copy(rows.at[0, pl.ds(0, size)],
                                acc.at[pl.ds(base + off, size)])

        def stage_x(base, total):
            @pl.when(cid == 0)
            def _():
                pltpu.sync_copy(x_hbm.at[pl.ds(base, total), pl.ds(0, DH)],
                                xspm.at[pl.ds(base, total)])

            @pl.when(cid == 1)
            def _():
                pltpu.sync_copy(x_hbm.at[pl.ds(base, total), pl.ds(DH, DH)],
                                xspm.at[pl.ds(base, total)])

        @pl.when(sid < NS - 1)
        def _():
            zero_acc(row0, RPS)
            stage_x(row0, RPS)

        @pl.when(sid == NS - 1)
        def _():
            zero_acc((NS - 1) * RPS, RPS_LAST)
            stage_x((NS - 1) * RPS, RPS_LAST)

        def idx_start(j, slot):
            pltpu.async_copy(ei_hbm.at[0, pl.ds(ebase + j * CHUNK, CHUNK)],
                             ibuf.at[slot, 0], sem_i)
            pltpu.async_copy(ei_hbm.at[1, pl.ds(ebase + j * CHUNK, CHUNK)],
                             ibuf.at[slot, 1], sem_i)

        def idx_wait():
            pltpu.make_async_copy(ei_hbm.at[0, pl.ds(0, CHUNK)],
                                  ibuf.at[0, 0], sem_i).wait()
            pltpu.make_async_copy(ei_hbm.at[1, pl.ds(0, CHUNK)],
                                  ibuf.at[0, 1], sem_i).wait()

        def gather_start(slot4, slot2):
            pltpu.async_copy(xspm.at[ibuf.at[slot4, 0]], rows.at[slot2],
                             sem_g)

        def gather_wait():
            pltpu.make_async_copy(xspm.at[ibuf.at[0, 0]], rows.at[0],
                                  sem_g).wait()

        pltpu.sync_copy(ei_hbm.at[0, pl.ds(ebase, CHUNK)], ibuf.at[0, 0])
        pltpu.sync_copy(ei_hbm.at[1, pl.ds(ebase, CHUNK)], ibuf.at[0, 1])
        plsc.subcore_barrier()
        gather_start(0, 0)
        idx_start(1, 1)

        def body(i, _):
            @pl.when(i + 1 < NCHUNK)
            def _():
                idx_wait()
                gather_start(lax.rem(i + 1, 4), lax.rem(i + 1, 2))

            @pl.when(i + 2 < NCHUNK)
            def _():
                idx_start(i + 2, lax.rem(i + 2, 4))

            gather_wait()
            pltpu.sync_copy(rows.at[lax.rem(i, 2)],
                            acc.at[ibuf.at[lax.rem(i, 4), 1]], add=True)
            return _

        lax.fori_loop(0, NCHUNK, body, None)

        plsc.subcore_barrier()

        out0 = pl.multiple_of(cid * N + sid * RPS, 8)

        @pl.when(sid < NS - 1)
        def _():
            pltpu.sync_copy(acc.at[pl.ds(row0, RPS)],
                            out_hbm.at[pl.ds(out0, RPS)])

        @pl.when(sid == NS - 1)
        def _():
            pltpu.sync_copy(
                acc.at[pl.ds((NS - 1) * RPS, RPS_LAST)],
                out_hbm.at[pl.ds(cid * N + (NS - 1) * RPS, RPS_LAST)],
            )

    return k(x, ei)


def _mm_kernel(pl_ref, pr_ref, wl_ref, wr_ref, b_ref, o_ref):
    y = jnp.dot(pl_ref[...], wl_ref[...], preferred_element_type=jnp.float32,
                precision=jax.lax.Precision.HIGHEST)
    y += jnp.dot(pr_ref[...], wr_ref[...], preferred_element_type=jnp.float32,
                 precision=jax.lax.Precision.HIGHEST)
    o_ref[...] = jnp.maximum(y + b_ref[...], 0.0)


def _finish_tc(partials, W, b2):
    blk = 2000
    nblk = N // blk
    return pl.pallas_call(
        _mm_kernel,
        grid=(nblk,),
        in_specs=[
            pl.BlockSpec((blk, DH), lambda i: (i, 0)),
            pl.BlockSpec((blk, DH), lambda i: (i + nblk, 0)),
            pl.BlockSpec((DH, D), lambda i: (0, 0)),
            pl.BlockSpec((DH, D), lambda i: (1, 0)),
            pl.BlockSpec((1, D), lambda i: (0, 0)),
        ],
        out_specs=pl.BlockSpec((blk, D), lambda i: (i, 0)),
        out_shape=jax.ShapeDtypeStruct((N, D), jnp.float32),
    )(partials, partials, W, W, b2)


def kernel(x, edge_index, W, b):
    partials = _segsum_sc(x, edge_index.astype(jnp.int32))
    return _finish_tc(partials, W, b.reshape(1, D))


# speedup vs baseline: 1.1959x; 1.1959x over previous
import functools

import jax
import jax.numpy as jnp
from jax import lax
from jax.experimental import pallas as pl
from jax.experimental.pallas import tpu as pltpu
from jax.experimental.pallas import tpu_sc as plsc

N = 10000
E = 320000
D = 128
DH = D // 2

NC = 2
NS = 16
EPS = E // NS
CHUNK = 128
NCHUNK = EPS // CHUNK
TAIL = EPS - NCHUNK * CHUNK

RPS = 632
RPS_LAST = N - 15 * RPS


def _segsum_sc(x, ei):
    mesh = plsc.VectorSubcoreMesh(core_axis_name="c", subcore_axis_name="s")

    @functools.partial(
        pl.kernel,
        mesh=mesh,
        compiler_params=pltpu.CompilerParams(use_tc_tiling_on_sc=False),
        out_type=jax.ShapeDtypeStruct((2 * N, DH), jnp.float32),
        scratch_types=[
            pltpu.VMEM((4, 2, CHUNK), jnp.int32),
            pltpu.VMEM((3, CHUNK, DH), jnp.float32),
            pltpu.VMEM_SHARED((N, DH), jnp.float32),
            pltpu.VMEM_SHARED((N + 8, DH), jnp.float32),
            pltpu.SemaphoreType.DMA,
            pltpu.SemaphoreType.DMA,
            pltpu.SemaphoreType.DMA,
        ],
    )
    def k(x_hbm, ei_hbm, out_hbm,
          ibuf, rows, xspm, acc, sem_i, sem_g, sem_s):
        cid = lax.axis_index("c")
        sid = lax.axis_index("s")
        ebase = sid * EPS

        zeros16 = jnp.zeros((16,), jnp.float32)

        def zero_body(t, _):
            rows[0, t // (DH // 16), pl.ds((t % (DH // 16)) * 16, 16)] = zeros16
            return _

        lax.fori_loop(0, CHUNK * (DH // 16), zero_body, None)
        row0 = pl.multiple_of(sid * RPS, 8)

        def zero_acc(base, total):
            for off in range(0, total, CHUNK):
                size = min(CHUNK, total - off)
                pltpu.sync_copy(rows.at[0, pl.ds(0, size)],
                                acc.at[pl.ds(base + off, size)])

        def stage_x(base, total):
            @pl.when(cid == 0)
            def _():
                pltpu.sync_copy(x_hbm.at[pl.ds(base, total), pl.ds(0, DH)],
                                xspm.at[pl.ds(base, total)])

            @pl.when(cid == 1)
            def _():
                pltpu.sync_copy(x_hbm.at[pl.ds(base, total), pl.ds(DH, DH)],
                                xspm.at[pl.ds(base, total)])

        @pl.when(sid < NS - 1)
        def _():
            zero_acc(row0, RPS)
            stage_x(row0, RPS)

        @pl.when(sid == NS - 1)
        def _():
            zero_acc((NS - 1) * RPS, RPS_LAST)
            stage_x((NS - 1) * RPS, RPS_LAST)

        def idx_start(j, slot):
            pltpu.async_copy(ei_hbm.at[0, pl.ds(ebase + j * CHUNK, CHUNK)],
                             ibuf.at[slot, 0], sem_i)
            pltpu.async_copy(ei_hbm.at[1, pl.ds(ebase + j * CHUNK, CHUNK)],
                             ibuf.at[slot, 1], sem_i)

        def idx_wait():
            pltpu.make_async_copy(ei_hbm.at[0, pl.ds(0, CHUNK)],
                                  ibuf.at[0, 0], sem_i).wait()
            pltpu.make_async_copy(ei_hbm.at[1, pl.ds(0, CHUNK)],
                                  ibuf.at[0, 1], sem_i).wait()

        def gather_start(slot4, slot3):
            pltpu.async_copy(xspm.at[ibuf.at[slot4, 0]], rows.at[slot3],
                             sem_g)

        def gather_wait():
            pltpu.make_async_copy(xspm.at[ibuf.at[0, 0]], rows.at[0],
                                  sem_g).wait()

        def scat_start(slot4, slot3):
            pltpu.async_copy(rows.at[slot3], acc.at[ibuf.at[slot4, 1]],
                             sem_s, add=True)

        def scat_wait():
            pltpu.make_async_copy(rows.at[0], acc.at[ibuf.at[0, 1]],
                                  sem_s).wait()

        pltpu.sync_copy(ei_hbm.at[0, pl.ds(ebase, CHUNK)], ibuf.at[0, 0])
        pltpu.sync_copy(ei_hbm.at[1, pl.ds(ebase, CHUNK)], ibuf.at[0, 1])
        plsc.subcore_barrier()
        gather_start(0, 0)
        idx_start(1, 1)

        def body(i, _):
            c4 = lax.rem(i, 4)
            c3 = lax.rem(i, 3)

            @pl.when(i >= 2)
            def _():
                scat_wait()

            @pl.when(i + 1 < NCHUNK)
            def _():
                idx_wait()
                gather_start(lax.rem(i + 1, 4), lax.rem(i + 1, 3))

            @pl.when(i + 2 < NCHUNK)
            def _():
                idx_start(i + 2, lax.rem(i + 2, 4))

            gather_wait()
            scat_start(c4, c3)
            return _

        lax.fori_loop(0, NCHUNK, body, None)
        scat_wait()
        scat_wait()

        tbase = ebase + NCHUNK * CHUNK
        pltpu.sync_copy(ei_hbm.at[0, pl.ds(tbase, TAIL)],
                        ibuf.at[0, 0, pl.ds(0, TAIL)])
        pltpu.sync_copy(ei_hbm.at[1, pl.ds(tbase, TAIL)],
                        ibuf.at[0, 1, pl.ds(0, TAIL)])
        for t in range((CHUNK - TAIL) // 16):
            ibuf[0, 0, pl.ds(TAIL + t * 16, 16)] = jnp.zeros((16,), jnp.int32)
            ibuf[0, 1, pl.ds(TAIL + t * 16, 16)] = jnp.full((16,), N,
                                                            jnp.int32)
        pltpu.async_copy(xspm.at[ibuf.at[0, 0]], rows.at[0], sem_g).wait()
        pltpu.sync_copy(rows.at[0], acc.at[ibuf.at[0, 1]], add=True)

        plsc.subcore_barrier()

        out0 = pl.multiple_of(cid * N + sid * RPS, 8)

        @pl.when(sid < NS - 1)
        def _():
            pltpu.sync_copy(acc.at[pl.ds(row0, RPS)],
                            out_hbm.at[pl.ds(out0, RPS)])

        @pl.when(sid == NS - 1)
        def _():
            pltpu.sync_copy(
                acc.at[pl.ds((NS - 1) * RPS, RPS_LAST)],
                out_hbm.at[pl.ds(cid * N + (NS - 1) * RPS, RPS_LAST)],
            )

    return k(x, ei)


def _mm_kernel(pl_ref, pr_ref, wl_ref, wr_ref, b_ref, o_ref):
    y = jnp.dot(pl_ref[...], wl_ref[...], preferred_element_type=jnp.float32,
                precision=jax.lax.Precision.HIGHEST)
    y += jnp.dot(pr_ref[...], wr_ref[...], preferred_element_type=jnp.float32,
                 precision=jax.lax.Precision.HIGHEST)
    o_ref[...] = jnp.maximum(y + b_ref[...], 0.0)


def _finish_tc(partials, W, b2):
    blk = 2000
    nblk = N // blk
    return pl.pallas_call(
        _mm_kernel,
        grid=(nblk,),
        in_specs=[
            pl.BlockSpec((blk, DH), lambda i: (i, 0)),
            pl.BlockSpec((blk, DH), lambda i: (i + nblk, 0)),
            pl.BlockSpec((DH, D), lambda i: (0, 0)),
            pl.BlockSpec((DH, D), lambda i: (1, 0)),
            pl.BlockSpec((1, D), lambda i: (0, 0)),
        ],
        out_specs=pl.BlockSpec((blk, D), lambda i: (i, 0)),
        out_shape=jax.ShapeDtypeStruct((N, D), jnp.float32),
    )(partials, partials, W, W, b2)


def kernel(x, edge_index, W, b):
    partials = _segsum_sc(x, edge_index.astype(jnp.int32))
    return _finish_tc(partials, W, b.reshape(1, D))


# speedup vs baseline: 1.1998x; 1.0032x over previous
import functools

import jax
import jax.numpy as jnp
from jax import lax
from jax.experimental import pallas as pl
from jax.experimental.pallas import tpu as pltpu
from jax.experimental.pallas import tpu_sc as plsc

N = 10000
E = 320000
D = 128
DH = D // 2

NC = 2
NS = 16
EPS = E // NS
CHUNK = 144
NCHUNK = EPS // CHUNK
TAIL = EPS - NCHUNK * CHUNK

RPS = 632
RPS_LAST = N - 15 * RPS


def _segsum_sc(x, ei):
    mesh = plsc.VectorSubcoreMesh(core_axis_name="c", subcore_axis_name="s")

    @functools.partial(
        pl.kernel,
        mesh=mesh,
        compiler_params=pltpu.CompilerParams(use_tc_tiling_on_sc=False),
        out_type=jax.ShapeDtypeStruct((2 * N, DH), jnp.float32),
        scratch_types=[
            pltpu.VMEM((4, 2, CHUNK), jnp.int32),
            pltpu.VMEM((3, CHUNK, DH), jnp.float32),
            pltpu.VMEM_SHARED((N, DH), jnp.float32),
            pltpu.VMEM_SHARED((N + 8, DH), jnp.float32),
            pltpu.SemaphoreType.DMA,
            pltpu.SemaphoreType.DMA,
            pltpu.SemaphoreType.DMA,
        ],
    )
    def k(x_hbm, ei_hbm, out_hbm,
          ibuf, rows, xspm, acc, sem_i, sem_g, sem_s):
        cid = lax.axis_index("c")
        sid = lax.axis_index("s")
        ebase = sid * EPS

        zeros16 = jnp.zeros((16,), jnp.float32)

        def zero_body(t, _):
            rows[0, t // (DH // 16), pl.ds((t % (DH // 16)) * 16, 16)] = zeros16
            return _

        lax.fori_loop(0, CHUNK * (DH // 16), zero_body, None)
        row0 = pl.multiple_of(sid * RPS, 8)

        def zero_acc(base, total):
            for off in range(0, total, CHUNK):
                size = min(CHUNK, total - off)
                pltpu.sync_copy(rows.at[0, pl.ds(0, size)],
                                acc.at[pl.ds(base + off, size)])

        def stage_x(base, total):
            @pl.when(cid == 0)
            def _():
                pltpu.sync_copy(x_hbm.at[pl.ds(base, total), pl.ds(0, DH)],
                                xspm.at[pl.ds(base, total)])

            @pl.when(cid == 1)
            def _():
                pltpu.sync_copy(x_hbm.at[pl.ds(base, total), pl.ds(DH, DH)],
                                xspm.at[pl.ds(base, total)])

        @pl.when(sid < NS - 1)
        def _():
            zero_acc(row0, RPS)
            stage_x(row0, RPS)

        @pl.when(sid == NS - 1)
        def _():
            zero_acc((NS - 1) * RPS, RPS_LAST)
            stage_x((NS - 1) * RPS, RPS_LAST)

        def idx_start(j, slot):
            pltpu.async_copy(ei_hbm.at[0, pl.ds(ebase + j * CHUNK, CHUNK)],
                             ibuf.at[slot, 0], sem_i)
            pltpu.async_copy(ei_hbm.at[1, pl.ds(ebase + j * CHUNK, CHUNK)],
                             ibuf.at[slot, 1], sem_i)

        def idx_wait():
            pltpu.make_async_copy(ei_hbm.at[0, pl.ds(0, CHUNK)],
                                  ibuf.at[0, 0], sem_i).wait()
            pltpu.make_async_copy(ei_hbm.at[1, pl.ds(0, CHUNK)],
                                  ibuf.at[0, 1], sem_i).wait()

        def gather_start(slot4, slot3):
            pltpu.async_copy(xspm.at[ibuf.at[slot4, 0]], rows.at[slot3],
                             sem_g)

        def gather_wait():
            pltpu.make_async_copy(xspm.at[ibuf.at[0, 0]], rows.at[0],
                                  sem_g).wait()

        def scat_start(slot4, slot3):
            pltpu.async_copy(rows.at[slot3], acc.at[ibuf.at[slot4, 1]],
                             sem_s, add=True)

        def scat_wait():
            pltpu.make_async_copy(rows.at[0], acc.at[ibuf.at[0, 1]],
                                  sem_s).wait()

        pltpu.sync_copy(ei_hbm.at[0, pl.ds(ebase, CHUNK)], ibuf.at[0, 0])
        pltpu.sync_copy(ei_hbm.at[1, pl.ds(ebase, CHUNK)], ibuf.at[0, 1])
        plsc.subcore_barrier()
        gather_start(0, 0)
        idx_start(1, 1)

        def body(i, _):
            c4 = lax.rem(i, 4)
            c3 = lax.rem(i, 3)

            @pl.when(i >= 2)
            def _():
                scat_wait()

            @pl.when(i + 1 < NCHUNK)
            def _():
                idx_wait()
                gather_start(lax.rem(i + 1, 4), lax.rem(i + 1, 3))

            @pl.when(i + 2 < NCHUNK)
            def _():
                idx_start(i + 2, lax.rem(i + 2, 4))

            gather_wait()
            scat_start(c4, c3)
            return _

        lax.fori_loop(0, NCHUNK, body, None)
        scat_wait()
        scat_wait()

        tbase = ebase + NCHUNK * CHUNK
        pltpu.sync_copy(ei_hbm.at[0, pl.ds(tbase, TAIL)],
                        ibuf.at[0, 0, pl.ds(0, TAIL)])
        pltpu.sync_copy(ei_hbm.at[1, pl.ds(tbase, TAIL)],
                        ibuf.at[0, 1, pl.ds(0, TAIL)])
        for t in range((CHUNK - TAIL) // 16):
            ibuf[0, 0, pl.ds(TAIL + t * 16, 16)] = jnp.zeros((16,), jnp.int32)
            ibuf[0, 1, pl.ds(TAIL + t * 16, 16)] = jnp.full((16,), N,
                                                            jnp.int32)
        pltpu.async_copy(xspm.at[ibuf.at[0, 0]], rows.at[0], sem_g).wait()
        pltpu.sync_copy(rows.at[0], acc.at[ibuf.at[0, 1]], add=True)

        plsc.subcore_barrier()

        out0 = pl.multiple_of(cid * N + sid * RPS, 8)

        @pl.when(sid < NS - 1)
        def _():
            pltpu.sync_copy(acc.at[pl.ds(row0, RPS)],
                            out_hbm.at[pl.ds(out0, RPS)])

        @pl.when(sid == NS - 1)
        def _():
            pltpu.sync_copy(
                acc.at[pl.ds((NS - 1) * RPS, RPS_LAST)],
                out_hbm.at[pl.ds(cid * N + (NS - 1) * RPS, RPS_LAST)],
            )

    return k(x, ei)


def _mm_kernel(pl_ref, pr_ref, wl_ref, wr_ref, b_ref, o_ref):
    y = jnp.dot(pl_ref[...], wl_ref[...], preferred_element_type=jnp.float32,
                precision=jax.lax.Precision.HIGHEST)
    y += jnp.dot(pr_ref[...], wr_ref[...], preferred_element_type=jnp.float32,
                 precision=jax.lax.Precision.HIGHEST)
    o_ref[...] = jnp.maximum(y + b_ref[...], 0.0)


def _finish_tc(partials, W, b2):
    blk = 2000
    nblk = N // blk
    return pl.pallas_call(
        _mm_kernel,
        grid=(nblk,),
        in_specs=[
            pl.BlockSpec((blk, DH), lambda i: (i, 0)),
            pl.BlockSpec((blk, DH), lambda i: (i + nblk, 0)),
            pl.BlockSpec((DH, D), lambda i: (0, 0)),
            pl.BlockSpec((DH, D), lambda i: (1, 0)),
            pl.BlockSpec((1, D), lambda i: (0, 0)),
        ],
        out_specs=pl.BlockSpec((blk, D), lambda i: (i, 0)),
        out_shape=jax.ShapeDtypeStruct((N, D), jnp.float32),
    )(partials, partials, W, W, b2)


def kernel(x, edge_index, W, b):
    partials = _segsum_sc(x, edge_index.astype(jnp.int32))
    return _finish_tc(partials, W, b.reshape(1, D))


# speedup vs baseline: 1.2303x; 1.0255x over previous
import functools

import jax
import jax.numpy as jnp
from jax import lax
from jax.experimental import pallas as pl
from jax.experimental.pallas import tpu as pltpu
from jax.experimental.pallas import tpu_sc as plsc

N = 10000
E = 320000
D = 128
DH = D // 2

NC = 2
NS = 16
EPS = E // NS
CHUNK = 144
NCHUNK = EPS // CHUNK
TAIL = EPS - NCHUNK * CHUNK

RPS = 632
RPS_LAST = N - 15 * RPS


def _segsum_sc(x, ei):
    mesh = plsc.VectorSubcoreMesh(core_axis_name="c", subcore_axis_name="s")

    @functools.partial(
        pl.kernel,
        mesh=mesh,
        compiler_params=pltpu.CompilerParams(use_tc_tiling_on_sc=False),
        out_type=jax.ShapeDtypeStruct((2 * N, DH), jnp.float32),
        scratch_types=[
            pltpu.VMEM((4, 2, CHUNK), jnp.int32),
            pltpu.VMEM((3, CHUNK, DH), jnp.float32),
            pltpu.VMEM_SHARED((N, DH), jnp.float32),
            pltpu.VMEM_SHARED((N + 8, DH), jnp.float32),
            pltpu.SemaphoreType.DMA,
            pltpu.SemaphoreType.DMA,
            pltpu.SemaphoreType.DMA,
        ],
    )
    def k(x_hbm, ei_hbm, out_hbm,
          ibuf, rows, xspm, acc, sem_i, sem_g, sem_s):
        cid = lax.axis_index("c")
        sid = lax.axis_index("s")
        ebase = sid * EPS

        zeros16 = jnp.zeros((16,), jnp.float32)

        def zero_body(t, _):
            rows[0, t // (DH // 16), pl.ds((t % (DH // 16)) * 16, 16)] = zeros16
            return _

        lax.fori_loop(0, CHUNK * (DH // 16), zero_body, None)
        row0 = pl.multiple_of(sid * RPS, 8)

        def zero_acc(base, total):
            for off in range(0, total, CHUNK):
                size = min(CHUNK, total - off)
                pltpu.sync_copy(rows.at[0, pl.ds(0, size)],
                                acc.at[pl.ds(base + off, size)])

        def stage_x(base, total):
            @pl.when(cid == 0)
            def _():
                pltpu.sync_copy(x_hbm.at[pl.ds(base, total), pl.ds(0, DH)],
                                xspm.at[pl.ds(base, total)])

            @pl.when(cid == 1)
            def _():
                pltpu.sync_copy(x_hbm.at[pl.ds(base, total), pl.ds(DH, DH)],
                                xspm.at[pl.ds(base, total)])

        @pl.when(sid < NS - 1)
        def _():
            zero_acc(row0, RPS)
            stage_x(row0, RPS)

        @pl.when(sid == NS - 1)
        def _():
            zero_acc((NS - 1) * RPS, RPS_LAST)
            stage_x((NS - 1) * RPS, RPS_LAST)

        def idx_start(j, slot):
            pltpu.async_copy(ei_hbm.at[0, pl.ds(ebase + j * CHUNK, CHUNK)],
                             ibuf.at[slot, 0], sem_i)
            pltpu.async_copy(ei_hbm.at[1, pl.ds(ebase + j * CHUNK, CHUNK)],
                             ibuf.at[slot, 1], sem_i)

        def idx_wait():
            pltpu.make_async_copy(ei_hbm.at[0, pl.ds(0, CHUNK)],
                                  ibuf.at[0, 0], sem_i).wait()
            pltpu.make_async_copy(ei_hbm.at[1, pl.ds(0, CHUNK)],
                                  ibuf.at[0, 1], sem_i).wait()

        def gather_start(slot4, slot3):
            pltpu.async_copy(xspm.at[ibuf.at[slot4, 0]], rows.at[slot3],
                             sem_g)

        def gather_wait():
            pltpu.make_async_copy(xspm.at[ibuf.at[0, 0]], rows.at[0],
                                  sem_g).wait()

        def scat_start(slot4, slot3):
            pltpu.async_copy(rows.at[slot3], acc.at[ibuf.at[slot4, 1]],
                             sem_s, add=True)

        def scat_wait():
            pltpu.make_async_copy(rows.at[0], acc.at[ibuf.at[0, 1]],
                                  sem_s).wait()

        pltpu.sync_copy(ei_hbm.at[0, pl.ds(ebase, CHUNK)], ibuf.at[0, 0])
        pltpu.sync_copy(ei_hbm.at[1, pl.ds(ebase, CHUNK)], ibuf.at[0, 1])
        plsc.subcore_barrier()
        gather_start(0, 0)
        idx_start(1, 1)

        def body(i, _):
            c4 = lax.rem(i, 4)
            c3 = lax.rem(i, 3)

            @pl.when(i >= 2)
            def _():
                scat_wait()

            @pl.when(i + 1 < NCHUNK)
            def _():
                idx_wait()
                gather_start(lax.rem(i + 1, 4), lax.rem(i + 1, 3))

            @pl.when(i + 2 < NCHUNK)
            def _():
                idx_start(i + 2, lax.rem(i + 2, 4))

            gather_wait()
            scat_start(c4, c3)
            return _

        lax.fori_loop(0, NCHUNK, body, None)
        scat_wait()
        scat_wait()

        tbase = ebase + NCHUNK * CHUNK
        pltpu.sync_copy(ei_hbm.at[0, pl.ds(tbase, TAIL)],
                        ibuf.at[0, 0, pl.ds(0, TAIL)])
        pltpu.sync_copy(ei_hbm.at[1, pl.ds(tbase, TAIL)],
                        ibuf.at[0, 1, pl.ds(0, TAIL)])
        for t in range((CHUNK - TAIL) // 16):
            ibuf[0, 0, pl.ds(TAIL + t * 16, 16)] = jnp.zeros((16,), jnp.int32)
            ibuf[0, 1, pl.ds(TAIL + t * 16, 16)] = jnp.full((16,), N,
                                                            jnp.int32)
        pltpu.async_copy(xspm.at[ibuf.at[0, 0]], rows.at[0], sem_g).wait()
        pltpu.sync_copy(rows.at[0], acc.at[ibuf.at[0, 1]], add=True)

        plsc.subcore_barrier()

        out0 = pl.multiple_of(cid * N + sid * RPS, 8)

        @pl.when(sid < NS - 1)
        def _():
            pltpu.sync_copy(acc.at[pl.ds(row0, RPS)],
                            out_hbm.at[pl.ds(out0, RPS)])

        @pl.when(sid == NS - 1)
        def _():
            pltpu.sync_copy(
                acc.at[pl.ds((NS - 1) * RPS, RPS_LAST)],
                out_hbm.at[pl.ds(cid * N + (NS - 1) * RPS, RPS_LAST)],
            )

    return k(x, ei)


def _mm_kernel(pl_ref, pr_ref, wl_ref, wr_ref, b_ref, o_ref):
    y = jnp.dot(pl_ref[...], wl_ref[...], preferred_element_type=jnp.float32)
    y += jnp.dot(pr_ref[...], wr_ref[...], preferred_element_type=jnp.float32)
    o_ref[...] = jnp.maximum(y + b_ref[...], 0.0)


def _finish_tc(partials, W, b2):
    blk = 2000
    nblk = N // blk
    return pl.pallas_call(
        _mm_kernel,
        grid=(nblk,),
        in_specs=[
            pl.BlockSpec((blk, DH), lambda i: (i, 0)),
            pl.BlockSpec((blk, DH), lambda i: (i + nblk, 0)),
            pl.BlockSpec((DH, D), lambda i: (0, 0)),
            pl.BlockSpec((DH, D), lambda i: (1, 0)),
            pl.BlockSpec((1, D), lambda i: (0, 0)),
        ],
        out_specs=pl.BlockSpec((blk, D), lambda i: (i, 0)),
        out_shape=jax.ShapeDtypeStruct((N, D), jnp.float32),
    )(partials, partials, W, W, b2)


def kernel(x, edge_index, W, b):
    partials = _segsum_sc(x, edge_index.astype(jnp.int32))
    return _finish_tc(partials, W, b.reshape(1, D))


# speedup vs baseline: 1.2507x; 1.0166x over previous
import functools

import jax
import jax.numpy as jnp
from jax import lax
from jax.experimental import pallas as pl
from jax.experimental.pallas import tpu as pltpu
from jax.experimental.pallas import tpu_sc as plsc

N = 10000
E = 320000
D = 128
DH = D // 2

NC = 2
NS = 16
EPS = E // NS
CHUNK = 144
NCHUNK = EPS // CHUNK
TAIL = EPS - NCHUNK * CHUNK

RPS = 632
RPS_LAST = N - 15 * RPS


def _segsum_sc(x, ei):
    mesh = plsc.VectorSubcoreMesh(core_axis_name="c", subcore_axis_name="s")

    @functools.partial(
        pl.kernel,
        mesh=mesh,
        compiler_params=pltpu.CompilerParams(use_tc_tiling_on_sc=False),
        out_type=jax.ShapeDtypeStruct((2 * N, DH), jnp.float32),
        scratch_types=[
            pltpu.VMEM((4, 2, CHUNK), jnp.int32),
            pltpu.VMEM((3, CHUNK, DH), jnp.float32),
            pltpu.VMEM_SHARED((N, DH), jnp.float32),
            pltpu.VMEM_SHARED((N + 8, DH), jnp.float32),
            pltpu.SemaphoreType.DMA,
            pltpu.SemaphoreType.DMA,
            pltpu.SemaphoreType.DMA,
        ],
    )
    def k(x_hbm, ei_hbm, out_hbm,
          ibuf, rows, xspm, acc, sem_i, sem_g, sem_s):
        cid = lax.axis_index("c")
        sid = lax.axis_index("s")
        ebase = sid * EPS

        zeros16 = jnp.zeros((16,), jnp.float32)

        def zero_body(t, _):
            rows[0, t // (DH // 16), pl.ds((t % (DH // 16)) * 16, 16)] = zeros16
            return _

        lax.fori_loop(0, CHUNK * (DH // 16), zero_body, None)
        row0 = pl.multiple_of(sid * RPS, 8)

        def zero_acc(base, total, issue):
            for off in range(0, total, CHUNK):
                size = min(CHUNK, total - off)
                cp = pltpu.make_async_copy(rows.at[0, pl.ds(0, size)],
                                           acc.at[pl.ds(base + off, size)],
                                           sem_s)
                cp.start() if issue else cp.wait()

        def stage_x(base, total, issue):
            @pl.when(cid == 0)
            def _():
                cp = pltpu.make_async_copy(
                    x_hbm.at[pl.ds(base, total), pl.ds(0, DH)],
                    xspm.at[pl.ds(base, total)], sem_s)
                cp.start() if issue else cp.wait()

            @pl.when(cid == 1)
            def _():
                cp = pltpu.make_async_copy(
                    x_hbm.at[pl.ds(base, total), pl.ds(DH, DH)],
                    xspm.at[pl.ds(base, total)], sem_s)
                cp.start() if issue else cp.wait()

        def prologue(issue):
            @pl.when(sid < NS - 1)
            def _():
                zero_acc(row0, RPS, issue)
                stage_x(row0, RPS, issue)

            @pl.when(sid == NS - 1)
            def _():
                zero_acc((NS - 1) * RPS, RPS_LAST, issue)
                stage_x((NS - 1) * RPS, RPS_LAST, issue)

        def idx_start(j, slot):
            pltpu.async_copy(ei_hbm.at[0, pl.ds(ebase + j * CHUNK, CHUNK)],
                             ibuf.at[slot, 0], sem_i)
            pltpu.async_copy(ei_hbm.at[1, pl.ds(ebase + j * CHUNK, CHUNK)],
                             ibuf.at[slot, 1], sem_i)

        def idx_wait():
            pltpu.make_async_copy(ei_hbm.at[0, pl.ds(0, CHUNK)],
                                  ibuf.at[0, 0], sem_i).wait()
            pltpu.make_async_copy(ei_hbm.at[1, pl.ds(0, CHUNK)],
                                  ibuf.at[0, 1], sem_i).wait()

        def gather_start(slot4, slot3):
            pltpu.async_copy(xspm.at[ibuf.at[slot4, 0]], rows.at[slot3],
                             sem_g)

        def gather_wait():
            pltpu.make_async_copy(xspm.at[ibuf.at[0, 0]], rows.at[0],
                                  sem_g).wait()

        def scat_start(slot4, slot3):
            pltpu.async_copy(rows.at[slot3], acc.at[ibuf.at[slot4, 1]],
                             sem_s, add=True)

        def scat_wait():
            pltpu.make_async_copy(rows.at[0], acc.at[ibuf.at[0, 1]],
                                  sem_s).wait()

        prologue(issue=True)
        idx_start(0, 0)
        prologue(issue=False)
        plsc.subcore_barrier()
        idx_wait()
        gather_start(0, 0)
        idx_start(1, 1)

        def body(i, _):
            c4 = lax.rem(i, 4)
            c3 = lax.rem(i, 3)

            @pl.when(i >= 2)
            def _():
                scat_wait()

            @pl.when(i + 1 < NCHUNK)
            def _():
                idx_wait()
                gather_start(lax.rem(i + 1, 4), lax.rem(i + 1, 3))

            @pl.when(i + 2 < NCHUNK)
            def _():
                idx_start(i + 2, lax.rem(i + 2, 4))

            gather_wait()
            scat_start(c4, c3)
            return _

        lax.fori_loop(0, NCHUNK, body, None)
        scat_wait()
        scat_wait()

        tbase = ebase + NCHUNK * CHUNK
        pltpu.sync_copy(ei_hbm.at[0, pl.ds(tbase, TAIL)],
                        ibuf.at[0, 0, pl.ds(0, TAIL)])
        pltpu.sync_copy(ei_hbm.at[1, pl.ds(tbase, TAIL)],
                        ibuf.at[0, 1, pl.ds(0, TAIL)])
        for t in range((CHUNK - TAIL) // 16):
            ibuf[0, 0, pl.ds(TAIL + t * 16, 16)] = jnp.zeros((16,), jnp.int32)
            ibuf[0, 1, pl.ds(TAIL + t * 16, 16)] = jnp.full((16,), N,
                                                            jnp.int32)
        pltpu.async_copy(xspm.at[ibuf.at[0, 0]], rows.at[0], sem_g).wait()
        pltpu.sync_copy(rows.at[0], acc.at[ibuf.at[0, 1]], add=True)

        plsc.subcore_barrier()

        out0 = pl.multiple_of(cid * N + sid * RPS, 8)

        @pl.when(sid < NS - 1)
        def _():
            pltpu.sync_copy(acc.at[pl.ds(row0, RPS)],
                            out_hbm.at[pl.ds(out0, RPS)])

        @pl.when(sid == NS - 1)
        def _():
            pltpu.sync_copy(
                acc.at[pl.ds((NS - 1) * RPS, RPS_LAST)],
                out_hbm.at[pl.ds(cid * N + (NS - 1) * RPS, RPS_LAST)],
            )

    return k(x, ei)


def _mm_kernel(pl_ref, pr_ref, wl_ref, wr_ref, b_ref, o_ref):
    y = jnp.dot(pl_ref[...], wl_ref[...], preferred_element_type=jnp.float32)
    y += jnp.dot(pr_ref[...], wr_ref[...], preferred_element_type=jnp.float32)
    o_ref[...] = jnp.maximum(y + b_ref[...], 0.0)


def _finish_tc(partials, W, b2):
    blk = 2000
    nblk = N // blk
    return pl.pallas_call(
        _mm_kernel,
        grid=(nblk,),
        in_specs=[
            pl.BlockSpec((blk, DH), lambda i: (i, 0)),
            pl.BlockSpec((blk, DH), lambda i: (i + nblk, 0)),
            pl.BlockSpec((DH, D), lambda i: (0, 0)),
            pl.BlockSpec((DH, D), lambda i: (1, 0)),
            pl.BlockSpec((1, D), lambda i: (0, 0)),
        ],
        out_specs=pl.BlockSpec((blk, D), lambda i: (i, 0)),
        out_shape=jax.ShapeDtypeStruct((N, D), jnp.float32),
    )(partials, partials, W, W, b2)


def kernel(x, edge_index, W, b):
    partials = _segsum_sc(x, edge_index.astype(jnp.int32))
    return _finish_tc(partials, W, b.reshape(1, D))


# speedup vs baseline: 1.2512x; 1.0004x over previous
import functools

import jax
import jax.numpy as jnp
from jax import lax
from jax.experimental import pallas as pl
from jax.experimental.pallas import tpu as pltpu
from jax.experimental.pallas import tpu_sc as plsc

N = 10000
E = 320000
D = 128
DH = D // 2

NC = 2
NS = 16
EPS = E // NS
CHUNK = 144
NCHUNK = EPS // CHUNK
TAIL = EPS - NCHUNK * CHUNK

RPS = 632
RPS_LAST = N - 15 * RPS


def _segsum_sc(x, ei):
    mesh = plsc.VectorSubcoreMesh(core_axis_name="c", subcore_axis_name="s")

    @functools.partial(
        pl.kernel,
        mesh=mesh,
        compiler_params=pltpu.CompilerParams(use_tc_tiling_on_sc=False),
        out_type=jax.ShapeDtypeStruct((2 * N, DH), jnp.float32),
        scratch_types=[
            pltpu.VMEM((4, 2, CHUNK), jnp.int32),
            pltpu.VMEM((3, CHUNK, DH), jnp.float32),
            pltpu.VMEM_SHARED((N, DH), jnp.float32),
            pltpu.VMEM_SHARED((N + 8, DH), jnp.float32),
            pltpu.SemaphoreType.DMA,
            pltpu.SemaphoreType.DMA,
            pltpu.SemaphoreType.DMA,
            pltpu.SemaphoreType.DMA,
            pltpu.SemaphoreType.DMA,
            pltpu.SemaphoreType.DMA,
        ],
    )
    def k(x_hbm, ei_hbm, out_hbm, ibuf, rows, xspm, acc,
          sem_i0, sem_i1, sem_g0, sem_g1, sem_s0, sem_s1):
        cid = lax.axis_index("c")
        sid = lax.axis_index("s")
        ebase = sid * EPS

        zeros16 = jnp.zeros((16,), jnp.float32)

        def zero_body(t, _):
            rows[0, t // (DH // 16), pl.ds((t % (DH // 16)) * 16, 16)] = zeros16
            return _

        lax.fori_loop(0, CHUNK * (DH // 16), zero_body, None)
        row0 = pl.multiple_of(sid * RPS, 8)

        def zero_acc(base, total, issue):
            for off in range(0, total, CHUNK):
                size = min(CHUNK, total - off)
                cp = pltpu.make_async_copy(rows.at[0, pl.ds(0, size)],
                                           acc.at[pl.ds(base + off, size)],
                                           sem_s0)
                cp.start() if issue else cp.wait()

        def stage_x(base, total, issue):
            @pl.when(cid == 0)
            def _():
                cp = pltpu.make_async_copy(
                    x_hbm.at[pl.ds(base, total), pl.ds(0, DH)],
                    xspm.at[pl.ds(base, total)], sem_s0)
                cp.start() if issue else cp.wait()

            @pl.when(cid == 1)
            def _():
                cp = pltpu.make_async_copy(
                    x_hbm.at[pl.ds(base, total), pl.ds(DH, DH)],
                    xspm.at[pl.ds(base, total)], sem_s0)
                cp.start() if issue else cp.wait()

        def prologue(issue):
            @pl.when(sid < NS - 1)
            def _():
                zero_acc(row0, RPS, issue)
                stage_x(row0, RPS, issue)

            @pl.when(sid == NS - 1)
            def _():
                zero_acc((NS - 1) * RPS, RPS_LAST, issue)
                stage_x((NS - 1) * RPS, RPS_LAST, issue)

        def sem_ip(p):
            return sem_i0 if p == 0 else sem_i1

        def sem_gp(p):
            return sem_g0 if p == 0 else sem_g1

        def sem_sp(p):
            return sem_s0 if p == 0 else sem_s1

        def idx_start(j, slot, p):
            pltpu.async_copy(ei_hbm.at[0, pl.ds(ebase + j * CHUNK, CHUNK)],
                             ibuf.at[slot, 0], sem_ip(p))
            pltpu.async_copy(ei_hbm.at[1, pl.ds(ebase + j * CHUNK, CHUNK)],
                             ibuf.at[slot, 1], sem_ip(p))

        def idx_wait(p):
            pltpu.make_async_copy(ei_hbm.at[0, pl.ds(0, CHUNK)],
                                  ibuf.at[0, 0], sem_ip(p)).wait()
            pltpu.make_async_copy(ei_hbm.at[1, pl.ds(0, CHUNK)],
                                  ibuf.at[0, 1], sem_ip(p)).wait()

        def gather_start(slot4, slot3, p):
            pltpu.async_copy(xspm.at[ibuf.at[slot4, 0]], rows.at[slot3],
                             sem_gp(p))

        def gather_wait(p):
            pltpu.make_async_copy(xspm.at[ibuf.at[0, 0]], rows.at[0],
                                  sem_gp(p)).wait()

        def scat_start(slot4, slot3, p):
            pltpu.async_copy(rows.at[slot3], acc.at[ibuf.at[slot4, 1]],
                             sem_sp(p), add=True)

        def scat_wait(p):
            pltpu.make_async_copy(rows.at[0], acc.at[ibuf.at[0, 1]],
                                  sem_sp(p)).wait()

        def step(i, p, w=True, g_next=True, i_next2=True):
            if w:
                scat_wait(p)
            if g_next:
                idx_wait(1 - p)
                gather_start(lax.rem(i + 1, 4), lax.rem(i + 1, 3), 1 - p)
            if i_next2:
                idx_start(i + 2, lax.rem(i + 2, 4), p)
            gather_wait(p)
            scat_start(lax.rem(i, 4), lax.rem(i, 3), p)

        prologue(issue=True)
        idx_start(0, 0, 0)
        prologue(issue=False)
        plsc.subcore_barrier()
        idx_wait(0)
        gather_start(0, 0, 0)
        idx_start(1, 1, 1)

        step(0, 0, w=False)
        step(1, 1, w=False)

        def body(i2, _):
            i = 2 * i2
            step(i, 0)
            step(i + 1, 1)
            return _

        lax.fori_loop(1, NCHUNK // 2 - 1, body, None)
        step(NCHUNK - 2, 0, i_next2=False)
        step(NCHUNK - 1, 1, g_next=False, i_next2=False)
        scat_wait(0)
        scat_wait(1)

        tbase = ebase + NCHUNK * CHUNK
        pltpu.sync_copy(ei_hbm.at[0, pl.ds(tbase, TAIL)],
                        ibuf.at[0, 0, pl.ds(0, TAIL)])
        pltpu.sync_copy(ei_hbm.at[1, pl.ds(tbase, TAIL)],
                        ibuf.at[0, 1, pl.ds(0, TAIL)])
        for t in range((CHUNK - TAIL) // 16):
            ibuf[0, 0, pl.ds(TAIL + t * 16, 16)] = jnp.zeros((16,), jnp.int32)
            ibuf[0, 1, pl.ds(TAIL + t * 16, 16)] = jnp.full((16,), N,
                                                            jnp.int32)
        pltpu.async_copy(xspm.at[ibuf.at[0, 0]], rows.at[0], sem_g0).wait()
        pltpu.sync_copy(rows.at[0], acc.at[ibuf.at[0, 1]], add=True)

        plsc.subcore_barrier()

        out0 = pl.multiple_of(cid * N + sid * RPS, 8)

        @pl.when(sid < NS - 1)
        def _():
            pltpu.sync_copy(acc.at[pl.ds(row0, RPS)],
                            out_hbm.at[pl.ds(out0, RPS)])

        @pl.when(sid == NS - 1)
        def _():
            pltpu.sync_copy(
                acc.at[pl.ds((NS - 1) * RPS, RPS_LAST)],
                out_hbm.at[pl.ds(cid * N + (NS - 1) * RPS, RPS_LAST)],
            )

    return k(x, ei)


def _mm_kernel(pl_ref, pr_ref, wl_ref, wr_ref, b_ref, o_ref):
    y = jnp.dot(pl_ref[...], wl_ref[...], preferred_element_type=jnp.float32)
    y += jnp.dot(pr_ref[...], wr_ref[...], preferred_element_type=jnp.float32)
    o_ref[...] = jnp.maximum(y + b_ref[...], 0.0)


def _finish_tc(partials, W, b2):
    blk = 2000
    nblk = N // blk
    return pl.pallas_call(
        _mm_kernel,
        grid=(nblk,),
        in_specs=[
            pl.BlockSpec((blk, DH), lambda i: (i, 0)),
            pl.BlockSpec((blk, DH), lambda i: (i + nblk, 0)),
            pl.BlockSpec((DH, D), lambda i: (0, 0)),
            pl.BlockSpec((DH, D), lambda i: (1, 0)),
            pl.BlockSpec((1, D), lambda i: (0, 0)),
        ],
        out_specs=pl.BlockSpec((blk, D), lambda i: (i, 0)),
        out_shape=jax.ShapeDtypeStruct((N, D), jnp.float32),
    )(partials, partials, W, W, b2)


def kernel(x, edge_index, W, b):
    partials = _segsum_sc(x, edge_index.astype(jnp.int32))
    return _finish_tc(partials, W, b.reshape(1, D))
